# baseline (speedup 1.0000x reference)
import functools

import jax
import jax.numpy as jnp
from jax.experimental import pallas as pl
from jax.experimental.pallas import tpu as pltpu

_MARGIN = 128


def _spiking_conv_kernel(x_ref, w_ref, b_ref, out_ref, scratch, *, C, H, W,
                         threshold, sim_length, chunk):
    HW = H * W
    scratch[:, :_MARGIN] = jnp.zeros((C, _MARGIN), jnp.bfloat16)
    scratch[:, _MARGIN:_MARGIN + HW] = x_ref[...]
    scratch[:, _MARGIN + HW:] = jnp.zeros((C, _MARGIN), jnp.bfloat16)

    col = jax.lax.broadcasted_iota(jnp.int32, (1, chunk), 1) % W
    mask_l = jnp.where(col == 0, 0.0, 1.0).astype(jnp.float32)
    mask_r = jnp.where(col == W - 1, 0.0, 1.0).astype(jnp.float32)

    thr = jnp.float32(threshold)
    scale = jnp.float32(float(sim_length) / float(threshold))
    wt = [w_ref[t].astype(jnp.bfloat16) for t in range(w_ref.shape[0])]

    for n0 in range(0, HW, chunk):
        def tap_sum(dw):
            acc = None
            for kh in range(3):
                t = kh * 3 + (dw + 1)
                off = _MARGIN + n0 + (kh - 1) * W + dw
                d = jnp.dot(wt[t], scratch[:, off:off + chunk],
                            preferred_element_type=jnp.float32)
                acc = d if acc is None else acc + d
            return acc

        acc = b_ref[...] + tap_sum(-1) * mask_l + tap_sum(0) \
            + tap_sum(1) * mask_r
        cnt = jnp.clip(jnp.floor(acc * scale), 0.0, jnp.float32(sim_length))
        out_ref[:, n0:n0 + chunk] = (thr * cnt).astype(jnp.bfloat16)


def kernel(x_nchw, weight, bias):
    threshold, sim_length = 1.0, 8
    B, C, H, W = x_nchw.shape
    OC, Cw, KH, KW = weight.shape
    HW = H * W

    x_flat = x_nchw.astype(jnp.bfloat16).reshape(B, C, HW)
    w_taps = weight.astype(jnp.float32).transpose(2, 3, 0, 1).reshape(
        KH * KW, OC, C)
    b_eff = (bias.astype(jnp.float32)
             + jnp.float32(threshold * 0.5 / sim_length)).reshape(OC, 1)

    kernel_fn = functools.partial(
        _spiking_conv_kernel, C=C, H=H, W=W, threshold=float(threshold),
        sim_length=int(sim_length), chunk=4096)

    cost = pl.CostEstimate(
        flops=B * (2 * KH * KW * OC * C * HW + 5 * OC * HW),
        transcendentals=0,
        bytes_accessed=4 * (B * C * HW + KH * KW * OC * C + OC + B * OC * HW),
    )

    out = pl.pallas_call(
        kernel_fn,
        out_shape=jax.ShapeDtypeStruct((B, OC, HW), jnp.bfloat16),
        grid=(B,),
        in_specs=[
            pl.BlockSpec((None, C, HW), lambda b: (b, 0, 0)),
            pl.BlockSpec((KH * KW, OC, C), lambda b: (0, 0, 0)),
            pl.BlockSpec((OC, 1), lambda b: (0, 0)),
        ],
        out_specs=pl.BlockSpec((None, OC, HW), lambda b: (b, 0, 0)),
        scratch_shapes=[pltpu.VMEM((C, 2 * _MARGIN + HW), jnp.bfloat16)],
        compiler_params=pltpu.CompilerParams(
            dimension_semantics=("parallel",),
        ),
        cost_estimate=cost,
    )(x_flat, w_taps, b_eff)

    return out.astype(jnp.float32).reshape(B, OC, H, W)


# speedup vs baseline: 1.1488x; 1.1488x over previous
import functools

import jax
import jax.numpy as jnp
from jax.experimental import pallas as pl
from jax.experimental.pallas import tpu as pltpu

_MARGIN = 128


def _spiking_conv_kernel(x_ref, w_ref, b_ref, out_ref, scratch, *, C, H, W,
                         threshold, sim_length, chunk):
    HW = H * W
    scratch[:, :_MARGIN] = jnp.zeros((C, _MARGIN), jnp.float32)
    scratch[:, _MARGIN:_MARGIN + HW] = x_ref[...]
    scratch[:, _MARGIN + HW:] = jnp.zeros((C, _MARGIN), jnp.float32)

    col = jax.lax.broadcasted_iota(jnp.int32, (1, chunk), 1) % W
    mask_l = jnp.where(col == 0, 0.0, 1.0).astype(jnp.float32)
    mask_r = jnp.where(col == W - 1, 0.0, 1.0).astype(jnp.float32)

    thr = jnp.float32(threshold)
    scale = jnp.float32(float(sim_length) / float(threshold))
    wt = [w_ref[t].astype(jnp.bfloat16) for t in range(w_ref.shape[0])]

    for n0 in range(0, HW, chunk):
        def tap_sum(dw):
            acc = None
            for kh in range(3):
                t = kh * 3 + (dw + 1)
                off = _MARGIN + n0 + (kh - 1) * W + dw
                patch = scratch[:, off:off + chunk].astype(jnp.bfloat16)
                d = jnp.dot(wt[t], patch,
                            preferred_element_type=jnp.float32)
                acc = d if acc is None else acc + d
            return acc

        acc = b_ref[...] + tap_sum(-1) * mask_l + tap_sum(0) \
            + tap_sum(1) * mask_r
        cnt = jnp.clip(jnp.floor(acc * scale), 0.0, jnp.float32(sim_length))
        out_ref[:, n0:n0 + chunk] = (thr * cnt).astype(jnp.bfloat16)


def kernel(x_nchw, weight, bias):
    threshold, sim_length = 1.0, 8
    B, C, H, W = x_nchw.shape
    OC, Cw, KH, KW = weight.shape
    HW = H * W

    x_flat = x_nchw.astype(jnp.float32).reshape(B, C, HW)
    w_taps = weight.astype(jnp.float32).transpose(2, 3, 0, 1).reshape(
        KH * KW, OC, C)
    b_eff = (bias.astype(jnp.float32)
             + jnp.float32(threshold * 0.5 / sim_length)).reshape(OC, 1)

    kernel_fn = functools.partial(
        _spiking_conv_kernel, C=C, H=H, W=W, threshold=float(threshold),
        sim_length=int(sim_length), chunk=4096)

    cost = pl.CostEstimate(
        flops=B * (2 * KH * KW * OC * C * HW + 5 * OC * HW),
        transcendentals=0,
        bytes_accessed=4 * (B * C * HW + KH * KW * OC * C + OC + B * OC * HW),
    )

    out = pl.pallas_call(
        kernel_fn,
        out_shape=jax.ShapeDtypeStruct((B, OC, HW), jnp.bfloat16),
        grid=(B,),
        in_specs=[
            pl.BlockSpec((None, C, HW), lambda b: (b, 0, 0)),
            pl.BlockSpec((KH * KW, OC, C), lambda b: (0, 0, 0)),
            pl.BlockSpec((OC, 1), lambda b: (0, 0)),
        ],
        out_specs=pl.BlockSpec((None, OC, HW), lambda b: (b, 0, 0)),
        scratch_shapes=[pltpu.VMEM((C, 2 * _MARGIN + HW), jnp.float32)],
        compiler_params=pltpu.CompilerParams(
            dimension_semantics=("parallel",),
        ),
        cost_estimate=cost,
    )(x_flat, w_taps, b_eff)

    return out.astype(jnp.float32).reshape(B, OC, H, W)
